# baseline (speedup 1.0000x reference)
import jax
import jax.numpy as jnp
from jax import lax
from jax.experimental import pallas as pl
from jax.experimental.pallas import tpu as pltpu
from jax.experimental.pallas import tpu_sc as plsc

N = 10000
E = 640000
NODE_DIM = 28
H = 128
NG = 50
NGP = 64
NGRAPHS = 16
NUM_INTER = 4
CUTOFF = 10.0

NC, NS = 2, 16
NW = NC * NS
CH = 64
CHUNKS = 320
EW = CH * CHUNKS
EP = NW * EW
PAD = EP - E
GCH = 128
NP = 10240
ZR = NP // NS
BE = 1024
NEB = EP // BE
BN = 1000
NNB = N // BN
BU = 80
NUB = N // BU

_DELTA = CUTOFF / (NG - 1)
_COEFF = -0.5 / (_DELTA * _DELTA)

_MESH = plsc.VectorSubcoreMesh(core_axis_name="c", subcore_axis_name="s",
                               num_cores=NC, num_subcores=NS)


def _silu(v):
    return v * jax.nn.sigmoid(v)


NPAD = 10240


def _geo_body(px, py, pz, rowp, colp, ew2b,
              pxv, pyv, pzv, rvs, cvs, obs, rss, css, oss):
    wid = lax.axis_index("s") * NC + lax.axis_index("c")
    base = wid * EW
    pltpu.sync_copy(px, pxv)
    pltpu.sync_copy(py, pyv)
    pltpu.sync_copy(pz, pzv)

    def si(t, k):
        pltpu.async_copy(rowp.at[pl.ds(base + t * GCH, GCH)], rvs[k], rss[k])
        pltpu.async_copy(colp.at[pl.ds(base + t * GCH, GCH)], cvs[k], css[k])

    def fin(t, k):
        pltpu.make_async_copy(rowp.at[pl.ds(base + t * GCH, GCH)],
                              rvs[k], rss[k]).wait()
        pltpu.make_async_copy(colp.at[pl.ds(base + t * GCH, GCH)],
                              cvs[k], css[k]).wait()

        @pl.when(t >= 2)
        def _wo():
            pltpu.make_async_copy(
                obs[k], ew2b.at[:, pl.ds(base + (t - 2) * GCH, GCH)],
                oss[k]).wait()

        for g in range(GCH // 16):
            sl = pl.ds(g * 16, 16)
            ri = rvs[k][sl]
            ci = cvs[k][sl]
            dx = plsc.load_gather(pxv, [ri]) - plsc.load_gather(pxv, [ci])
            dy = plsc.load_gather(pyv, [ri]) - plsc.load_gather(pyv, [ci])
            dz = plsc.load_gather(pzv, [ri]) - plsc.load_gather(pzv, [ci])
            e2 = dx * dx + dy * dy + dz * dz
            for r in range(8):
                obs[k][r, sl] = e2
        pltpu.async_copy(obs[k], ew2b.at[:, pl.ds(base + t * GCH, GCH)],
                         oss[k])

    si(0, 0)
    si(1, 1)

    def step(u, _):
        t0 = u * 2
        fin(t0, 0)

        @pl.when(t0 + 2 < EW // GCH)
        def _p0():
            si(t0 + 2, 0)

        fin(t0 + 1, 1)

        @pl.when(t0 + 3 < EW // GCH)
        def _p1():
            si(t0 + 3, 1)

        return _

    lax.fori_loop(0, EW // GCH // 2, step, 0)
    nt = EW // GCH
    pltpu.make_async_copy(obs[0], ew2b.at[:, pl.ds(base + (nt - 2) * GCH,
                                                   GCH)], oss[0]).wait()
    pltpu.make_async_copy(obs[1], ew2b.at[:, pl.ds(base + (nt - 1) * GCH,
                                                   GCH)], oss[1]).wait()


def _sc_geo(px, py, pz, rowp, colp):
    return pl.kernel(
        _geo_body,
        out_type=jax.ShapeDtypeStruct((8, EP), jnp.float32),
        mesh=_MESH,
        scratch_types=[
            pltpu.VMEM((NPAD,), jnp.float32),
            pltpu.VMEM((NPAD,), jnp.float32),
            pltpu.VMEM((NPAD,), jnp.float32),
            [pltpu.VMEM((GCH,), jnp.int32) for _ in range(2)],
            [pltpu.VMEM((GCH,), jnp.int32) for _ in range(2)],
            [pltpu.VMEM((8, GCH), jnp.float32) for _ in range(2)],
            [pltpu.SemaphoreType.DMA for _ in range(2)],
            [pltpu.SemaphoreType.DMA for _ in range(2)],
            [pltpu.SemaphoreType.DMA for _ in range(2)],
        ],
        compiler_params=pltpu.CompilerParams(needs_layout_passes=False),
    )(px, py, pz, rowp, colp)


def _attr_body(ew2_ref, out_ref):
    ew = jnp.sqrt(ew2_ref[0:1, :])
    ki = lax.broadcasted_iota(jnp.int32, (NGP, BE), 0)
    dd = ew - ki.astype(jnp.float32) * _DELTA
    mask = ki < NG
    out_ref[...] = jnp.where(mask, jnp.exp(_COEFF * dd * dd),
                             0.0).astype(jnp.bfloat16)


def _tc_attr(ew2b):
    return pl.pallas_call(
        _attr_body,
        grid=(NEB,),
        in_specs=[pl.BlockSpec((8, BE), lambda e: (0, e))],
        out_specs=pl.BlockSpec((NGP, BE), lambda e: (0, e)),
        out_shape=jax.ShapeDtypeStruct((NGP, EP), jnp.bfloat16),
    )(ew2b)


def _filter_body(attr_ref, w1_ref, b1_ref, w2_ref, b2_ref, out_ref):
    at = attr_ref[...]
    z = lax.dot_general(at, w1_ref[...], (((0,), (0,)), ((), ())),
                        preferred_element_type=jnp.float32)
    z = _silu(z + b1_ref[...])
    out_ref[...] = jnp.dot(z.astype(jnp.bfloat16), w2_ref[...],
                           preferred_element_type=jnp.float32) + b2_ref[...]


def _tc_filter(eattr, fn1p_l, fn1_b_l, fn2_W_l, fn2_b_l):
    return pl.pallas_call(
        _filter_body,
        grid=(NEB,),
        in_specs=[
            pl.BlockSpec((NGP, BE), lambda e: (0, e)),
            pl.BlockSpec((NGP, H), lambda e: (0, 0)),
            pl.BlockSpec((1, H), lambda e: (0, 0)),
            pl.BlockSpec((H, H), lambda e: (0, 0)),
            pl.BlockSpec((1, H), lambda e: (0, 0)),
        ],
        out_specs=pl.BlockSpec((BE, H), lambda e: (e, 0)),
        out_shape=jax.ShapeDtypeStruct((EP, H), jnp.float32),
    )(eattr, fn1p_l, fn1_b_l[None, :], fn2_W_l, fn2_b_l[None, :])


def _emb_body(x_ref, we_ref, be_ref, l1_ref, h_ref, hx_ref):
    h = jnp.dot(x_ref[...], we_ref[...],
                preferred_element_type=jnp.float32) + be_ref[...]
    h_ref[...] = h
    hx_ref[...] = jnp.dot(h, l1_ref[...], preferred_element_type=jnp.float32)


def _tc_emb(x, W_emb, b_emb, lin1_0):
    return pl.pallas_call(
        _emb_body,
        grid=(NNB,),
        in_specs=[
            pl.BlockSpec((BN, NODE_DIM), lambda i: (i, 0)),
            pl.BlockSpec((NODE_DIM, H), lambda i: (0, 0)),
            pl.BlockSpec((1, H), lambda i: (0, 0)),
            pl.BlockSpec((H, H), lambda i: (0, 0)),
        ],
        out_specs=[pl.BlockSpec((BN, H), lambda i: (i, 0)),
                   pl.BlockSpec((BN, H), lambda i: (i, 0))],
        out_shape=[jax.ShapeDtypeStruct((N, H), jnp.float32),
                   jax.ShapeDtypeStruct((N, H), jnp.float32)],
    )(x, W_emb, b_emb[None, :], lin1_0)


def _msg_body(hx, wf, rowp, colp, zrows, agg2,
             rows, cols, ghxs, wfvs, rss, css, gss, wss, agg_sh):
        c = lax.axis_index("c")
        s = lax.axis_index("s")
        wid = s * NC + c
        base = wid * EW

        pltpu.sync_copy(zrows, agg_sh.at[pl.ds(s * ZR, ZR)])
        plsc.subcore_barrier()

        def si(t, k):
            pltpu.async_copy(rowp.at[pl.ds(base + t * CH, CH)], rows[k],
                             rss[k])
            pltpu.async_copy(colp.at[pl.ds(base + t * CH, CH)], cols[k],
                             css[k])

        def sg(t, k, b):
            pltpu.make_async_copy(rowp.at[pl.ds(base + t * CH, CH)],
                                  rows[k], rss[k]).wait()
            pltpu.async_copy(hx.at[rows[k]], ghxs[b], gss[b])
            pltpu.async_copy(wf.at[pl.ds(base + t * CH, CH)],
                             wfvs[b], wss[b])

        def fin(t, k, b):
            ghx, wfv = ghxs[b], wfvs[b]
            pltpu.make_async_copy(hx.at[rows[k]], ghx, gss[b]).wait()
            pltpu.make_async_copy(wf.at[pl.ds(base + t * CH, CH)],
                                  wfv, wss[b]).wait()
            pltpu.make_async_copy(colp.at[pl.ds(base + t * CH, CH)],
                                  cols[k], css[k]).wait()

            @plsc.parallel_loop(0, CH, 1, unroll=4)
            def _mul(r):
                for j in range(H // 16):
                    sl = pl.ds(j * 16, 16)
                    ghx[r, sl] = ghx[r, sl] * wfv[r, sl]

            pltpu.sync_copy(ghx, agg_sh.at[cols[k]], add=True)

        si(0, 0)
        si(1, 1)
        sg(0, 0, 0)

        def step(q, _):
            t0 = q * 4
            for kk in range(4):
                t = t0 + kk

                @pl.when(t + 1 < CHUNKS)
                def _nx():
                    sg(t + 1, (kk + 1) % 4, (kk + 1) % 2)

                @pl.when(t + 2 < CHUNKS)
                def _pf():
                    si(t + 2, (kk + 2) % 4)

                fin(t, kk, kk % 2)
            return _

        lax.fori_loop(0, CHUNKS // 4, step, 0)
        plsc.subcore_barrier()
        pltpu.sync_copy(agg_sh.at[pl.ds(s * ZR, ZR)],
                        agg2.at[pl.ds(c * NP + s * ZR, ZR)])


def _sc_msg(hx, wf, rowp, colp, zrows):
    return pl.kernel(
        _msg_body,
        out_type=jax.ShapeDtypeStruct((NC * NP, H), jnp.float32),
        mesh=_MESH,
        scratch_types=[
            [pltpu.VMEM((CH,), jnp.int32) for _ in range(4)],
            [pltpu.VMEM((CH,), jnp.int32) for _ in range(4)],
            [pltpu.VMEM((CH, H), jnp.float32) for _ in range(2)],
            [pltpu.VMEM((CH, H), jnp.float32) for _ in range(2)],
            [pltpu.SemaphoreType.DMA for _ in range(4)],
            [pltpu.SemaphoreType.DMA for _ in range(4)],
            [pltpu.SemaphoreType.DMA for _ in range(2)],
            [pltpu.SemaphoreType.DMA for _ in range(2)],
            pltpu.VMEM_SHARED((NP, H), jnp.float32),
        ],
        cost_estimate=pl.CostEstimate(
            flops=2 * EP * H,
            bytes_accessed=EP * (8 * H + 16) + 3 * NC * NP * H * 4,
            transcendentals=0,
        ),
    )(hx, wf, rowp, colp, zrows)


def _upd_body(a0_ref, a1_ref, h_ref, l2_ref, b2_ref, l1n_ref,
              hn_ref, hxn_ref):
    agg = a0_ref[...] + a1_ref[...]
    hn = h_ref[...] + jnp.dot(agg, l2_ref[...],
                              preferred_element_type=jnp.float32) + b2_ref[...]
    hn_ref[...] = hn
    hxn_ref[...] = jnp.dot(hn, l1n_ref[...],
                           preferred_element_type=jnp.float32)


def _tc_update(agg2, h, lin2_i, lin2_b_i, lin1_next):
    return pl.pallas_call(
        _upd_body,
        grid=(NUB,),
        in_specs=[
            pl.BlockSpec((BU, H), lambda i: (i, 0)),
            pl.BlockSpec((BU, H), lambda i: (i + NP // BU, 0)),
            pl.BlockSpec((BU, H), lambda i: (i, 0)),
            pl.BlockSpec((H, H), lambda i: (0, 0)),
            pl.BlockSpec((1, H), lambda i: (0, 0)),
            pl.BlockSpec((H, H), lambda i: (0, 0)),
        ],
        out_specs=[pl.BlockSpec((BU, H), lambda i: (i, 0)),
                   pl.BlockSpec((BU, H), lambda i: (i, 0))],
        out_shape=[jax.ShapeDtypeStruct((N, H), jnp.float32),
                   jax.ShapeDtypeStruct((N, H), jnp.float32)],
    )(agg2, agg2, h, lin2_i, lin2_b_i[None, :], lin1_next)


def _pool_body(batch_ref, h_ref, e3_ref, w1a_ref, w1b_ref, b1_ref,
               w2_ref, b2_ref, w3_ref, b3_ref, out_ref, acc_ref, cnt_ref):
    i = pl.program_id(0)
    nb = pl.num_programs(0)

    @pl.when(i == 0)
    def _init():
        acc_ref[...] = jnp.zeros_like(acc_ref)
        cnt_ref[...] = jnp.zeros_like(cnt_ref)

    b = batch_ref[0, 0, :]
    gids = lax.broadcasted_iota(jnp.int32, (NGRAPHS, BN), 0)
    onehot = (b[None, :] == gids).astype(jnp.float32)
    acc_ref[...] += jnp.dot(onehot, h_ref[...],
                            preferred_element_type=jnp.float32)
    cnt_ref[...] += jnp.sum(onehot, axis=1)[None, :]

    @pl.when(i == nb - 1)
    def _final():
        counts = jnp.maximum(cnt_ref[0, :], 1.0)
        scale = 1.0 / (counts * jnp.sqrt(counts))
        pooled = acc_ref[...] * scale[:, None]
        o1 = _silu(pooled @ w1a_ref[...] + e3_ref[...] @ w1b_ref[...]
                   + b1_ref[...])
        o2 = _silu(o1 @ w2_ref[...] + b2_ref[...])
        out_ref[...] = o2 @ w3_ref[...] + b3_ref[...]


def _tc_pool(h, batch, e3_row, out1_W, out1_b, out2_W, out2_b,
             out3_W, out3_b):
    batch3 = batch.reshape(NNB, 1, BN)
    return pl.pallas_call(
        _pool_body,
        grid=(NNB,),
        in_specs=[
            pl.BlockSpec((1, 1, BN), lambda i: (i, 0, 0)),
            pl.BlockSpec((BN, H), lambda i: (i, 0)),
            pl.BlockSpec((1, H), lambda i: (0, 0)),
            pl.BlockSpec((H, H), lambda i: (0, 0)),
            pl.BlockSpec((H, H), lambda i: (0, 0)),
            pl.BlockSpec((1, H), lambda i: (0, 0)),
            pl.BlockSpec((H, H // 2), lambda i: (0, 0)),
            pl.BlockSpec((1, H // 2), lambda i: (0, 0)),
            pl.BlockSpec((H // 2, 1), lambda i: (0, 0)),
            pl.BlockSpec((1, 1), lambda i: (0, 0)),
        ],
        out_specs=pl.BlockSpec((NGRAPHS, 1), lambda i: (0, 0)),
        out_shape=jax.ShapeDtypeStruct((NGRAPHS, 1), jnp.float32),
        scratch_shapes=[
            pltpu.VMEM((NGRAPHS, H), jnp.float32),
            pltpu.VMEM((1, NGRAPHS), jnp.float32),
        ],
    )(batch3, h, e3_row, out1_W[:H], out1_W[H:], out1_b[None, :],
      out2_W, out2_b[None, :], out3_W, out3_b[None, :])


def kernel(x, pos, edge_index, batch, e3_idx, W_emb, b_emb, lin1_W, lin2_W,
           lin2_b, fn1_W, fn1_b, fn2_W, fn2_b, e3_table, out1_W, out1_b,
           out2_W, out2_b, out3_W, out3_b):
    row = edge_index[0]
    col = edge_index[1]
    pad_i = jnp.arange(PAD, dtype=jnp.int32)
    rowp = jnp.concatenate([row, pad_i % N])
    colp = jnp.concatenate([col, N + (pad_i % (NP - N))])
    px = jnp.pad(pos[:, 0], (0, NPAD - N))
    py = jnp.pad(pos[:, 1], (0, NPAD - N))
    pz = jnp.pad(pos[:, 2], (0, NPAD - N))
    fn1p = jnp.pad(fn1_W, ((0, 0), (0, NGP - NG), (0, 0))).astype(jnp.bfloat16)
    fn2b16 = fn2_W.astype(jnp.bfloat16)
    zrows = jnp.zeros((ZR, H), jnp.float32)

    ew2b = _sc_geo(px, py, pz, rowp, colp)
    eattr = _tc_attr(ew2b)
    wfs = [_tc_filter(eattr, fn1p[i], fn1_b[i], fn2b16[i], fn2_b[i])
           for i in range(NUM_INTER)]

    h, hx = _tc_emb(x, W_emb, b_emb, lin1_W[0])
    for i in range(NUM_INTER):
        agg2 = _sc_msg(hx, wfs[i], rowp, colp, zrows)
        lin1_next = lin1_W[(i + 1) % NUM_INTER]
        h, hx = _tc_update(agg2, h, lin2_W[i], lin2_b[i], lin1_next)

    e3_row = e3_table[e3_idx][None, :]
    return _tc_pool(h, batch, e3_row, out1_W, out1_b, out2_W, out2_b,
                    out3_W, out3_b)


# speedup vs baseline: 1.0364x; 1.0364x over previous
import jax
import jax.numpy as jnp
from jax import lax
from jax.experimental import pallas as pl
from jax.experimental.pallas import tpu as pltpu
from jax.experimental.pallas import tpu_sc as plsc

N = 10000
E = 640000
NODE_DIM = 28
H = 128
NG = 50
NGP = 64
NGRAPHS = 16
NUM_INTER = 4
CUTOFF = 10.0

NC, NS = 2, 16
NW = NC * NS
CH = 64
CHUNKS = 320
EW = CH * CHUNKS
EP = NW * EW
PAD = EP - E
GCH = 128
NP = 10240
ZR = NP // NS
BE = 1024
NEB = EP // BE
BN = 1000
NNB = N // BN
BU = 80
NUB = N // BU

_DELTA = CUTOFF / (NG - 1)
_COEFF = -0.5 / (_DELTA * _DELTA)

_MESH = plsc.VectorSubcoreMesh(core_axis_name="c", subcore_axis_name="s",
                               num_cores=NC, num_subcores=NS)


def _silu(v):
    return v * jax.nn.sigmoid(v)


NPAD = 10240


def _geo_body(px, py, pz, rowp, colp, ew2b,
              pxv, pyv, pzv, rvs, cvs, obs, rss, css, oss):
    wid = lax.axis_index("s") * NC + lax.axis_index("c")
    base = wid * EW
    pltpu.sync_copy(px, pxv)
    pltpu.sync_copy(py, pyv)
    pltpu.sync_copy(pz, pzv)

    def si(t, k):
        pltpu.async_copy(rowp.at[pl.ds(base + t * GCH, GCH)], rvs[k], rss[k])
        pltpu.async_copy(colp.at[pl.ds(base + t * GCH, GCH)], cvs[k], css[k])

    def fin(t, k):
        pltpu.make_async_copy(rowp.at[pl.ds(base + t * GCH, GCH)],
                              rvs[k], rss[k]).wait()
        pltpu.make_async_copy(colp.at[pl.ds(base + t * GCH, GCH)],
                              cvs[k], css[k]).wait()

        @pl.when(t >= 2)
        def _wo():
            pltpu.make_async_copy(
                obs[k], ew2b.at[:, pl.ds(base + (t - 2) * GCH, GCH)],
                oss[k]).wait()

        for g in range(GCH // 16):
            sl = pl.ds(g * 16, 16)
            ri = rvs[k][sl]
            ci = cvs[k][sl]
            dx = plsc.load_gather(pxv, [ri]) - plsc.load_gather(pxv, [ci])
            dy = plsc.load_gather(pyv, [ri]) - plsc.load_gather(pyv, [ci])
            dz = plsc.load_gather(pzv, [ri]) - plsc.load_gather(pzv, [ci])
            e2 = dx * dx + dy * dy + dz * dz
            for r in range(8):
                obs[k][r, sl] = e2
        pltpu.async_copy(obs[k], ew2b.at[:, pl.ds(base + t * GCH, GCH)],
                         oss[k])

    si(0, 0)
    si(1, 1)

    def step(u, _):
        t0 = u * 2
        fin(t0, 0)

        @pl.when(t0 + 2 < EW // GCH)
        def _p0():
            si(t0 + 2, 0)

        fin(t0 + 1, 1)

        @pl.when(t0 + 3 < EW // GCH)
        def _p1():
            si(t0 + 3, 1)

        return _

    lax.fori_loop(0, EW // GCH // 2, step, 0)
    nt = EW // GCH
    pltpu.make_async_copy(obs[0], ew2b.at[:, pl.ds(base + (nt - 2) * GCH,
                                                   GCH)], oss[0]).wait()
    pltpu.make_async_copy(obs[1], ew2b.at[:, pl.ds(base + (nt - 1) * GCH,
                                                   GCH)], oss[1]).wait()


def _sc_geo(px, py, pz, rowp, colp):
    return pl.kernel(
        _geo_body,
        out_type=jax.ShapeDtypeStruct((8, EP), jnp.float32),
        mesh=_MESH,
        scratch_types=[
            pltpu.VMEM((NPAD,), jnp.float32),
            pltpu.VMEM((NPAD,), jnp.float32),
            pltpu.VMEM((NPAD,), jnp.float32),
            [pltpu.VMEM((GCH,), jnp.int32) for _ in range(2)],
            [pltpu.VMEM((GCH,), jnp.int32) for _ in range(2)],
            [pltpu.VMEM((8, GCH), jnp.float32) for _ in range(2)],
            [pltpu.SemaphoreType.DMA for _ in range(2)],
            [pltpu.SemaphoreType.DMA for _ in range(2)],
            [pltpu.SemaphoreType.DMA for _ in range(2)],
        ],
        compiler_params=pltpu.CompilerParams(needs_layout_passes=False),
    )(px, py, pz, rowp, colp)


def _attr_body(ew2_ref, out_ref):
    ew2 = jnp.transpose(ew2_ref[...])[:, 0:1]
    ew = jnp.sqrt(ew2)
    ki = lax.broadcasted_iota(jnp.int32, (BE, NGP), 1)
    dd = ew - ki.astype(jnp.float32) * _DELTA
    mask = ki < NG
    out_ref[...] = jnp.where(mask, jnp.exp(_COEFF * dd * dd),
                             0.0).astype(jnp.bfloat16)


def _tc_attr(ew2b):
    return pl.pallas_call(
        _attr_body,
        grid=(NEB,),
        in_specs=[pl.BlockSpec((8, BE), lambda e: (0, e))],
        out_specs=pl.BlockSpec((BE, NGP), lambda e: (e, 0)),
        out_shape=jax.ShapeDtypeStruct((EP, NGP), jnp.bfloat16),
    )(ew2b)


def _filter_body(attr_ref, w1_ref, b1_ref, w2_ref, b2_ref, out_ref):
    a = attr_ref[...]
    z = jnp.dot(a, w1_ref[...], preferred_element_type=jnp.float32)
    z = _silu(z + b1_ref[...])
    out_ref[...] = jnp.dot(z.astype(jnp.bfloat16), w2_ref[...],
                           preferred_element_type=jnp.float32) + b2_ref[...]


def _tc_filter(eattr, fn1p_l, fn1_b_l, fn2_W_l, fn2_b_l):
    return pl.pallas_call(
        _filter_body,
        grid=(NEB,),
        in_specs=[
            pl.BlockSpec((BE, NGP), lambda e: (e, 0)),
            pl.BlockSpec((NGP, H), lambda e: (0, 0)),
            pl.BlockSpec((1, H), lambda e: (0, 0)),
            pl.BlockSpec((H, H), lambda e: (0, 0)),
            pl.BlockSpec((1, H), lambda e: (0, 0)),
        ],
        out_specs=pl.BlockSpec((BE, H), lambda e: (e, 0)),
        out_shape=jax.ShapeDtypeStruct((EP, H), jnp.float32),
    )(eattr, fn1p_l, fn1_b_l[None, :], fn2_W_l, fn2_b_l[None, :])


def _emb_body(x_ref, we_ref, be_ref, l1_ref, h_ref, hx_ref):
    h = jnp.dot(x_ref[...], we_ref[...],
                preferred_element_type=jnp.float32) + be_ref[...]
    h_ref[...] = h
    hx_ref[...] = jnp.dot(h, l1_ref[...], preferred_element_type=jnp.float32)


def _tc_emb(x, W_emb, b_emb, lin1_0):
    return pl.pallas_call(
        _emb_body,
        grid=(NNB,),
        in_specs=[
            pl.BlockSpec((BN, NODE_DIM), lambda i: (i, 0)),
            pl.BlockSpec((NODE_DIM, H), lambda i: (0, 0)),
            pl.BlockSpec((1, H), lambda i: (0, 0)),
            pl.BlockSpec((H, H), lambda i: (0, 0)),
        ],
        out_specs=[pl.BlockSpec((BN, H), lambda i: (i, 0)),
                   pl.BlockSpec((BN, H), lambda i: (i, 0))],
        out_shape=[jax.ShapeDtypeStruct((N, H), jnp.float32),
                   jax.ShapeDtypeStruct((N, H), jnp.float32)],
    )(x, W_emb, b_emb[None, :], lin1_0)


def _msg_body(hx, wf, rowp, colp, zrows, agg0, agg1,
             rows, cols, ghxs, wfvs, rss, css, gss, wss, agg_sh):
        c = lax.axis_index("c")
        s = lax.axis_index("s")
        wid = s * NC + c
        base = wid * EW

        pltpu.sync_copy(zrows, agg_sh.at[pl.ds(s * ZR, ZR)])
        plsc.subcore_barrier()

        def si(t, k):
            pltpu.async_copy(rowp.at[pl.ds(base + t * CH, CH)], rows[k],
                             rss[k])
            pltpu.async_copy(colp.at[pl.ds(base + t * CH, CH)], cols[k],
                             css[k])

        def sg(t, k, b):
            pltpu.make_async_copy(rowp.at[pl.ds(base + t * CH, CH)],
                                  rows[k], rss[k]).wait()
            pltpu.async_copy(hx.at[rows[k]], ghxs[b], gss[b])
            pltpu.async_copy(wf.at[pl.ds(base + t * CH, CH)],
                             wfvs[b], wss[b])

        def fin(t, k, b):
            ghx, wfv = ghxs[b], wfvs[b]
            pltpu.make_async_copy(hx.at[rows[k]], ghx, gss[b]).wait()
            pltpu.make_async_copy(wf.at[pl.ds(base + t * CH, CH)],
                                  wfv, wss[b]).wait()
            pltpu.make_async_copy(colp.at[pl.ds(base + t * CH, CH)],
                                  cols[k], css[k]).wait()

            @plsc.parallel_loop(0, CH, 1, unroll=4)
            def _mul(r):
                for j in range(H // 16):
                    sl = pl.ds(j * 16, 16)
                    ghx[r, sl] = ghx[r, sl] * wfv[r, sl]

            pltpu.sync_copy(ghx, agg_sh.at[cols[k]], add=True)

        si(0, 0)
        si(1, 1)
        sg(0, 0, 0)

        def step(q, _):
            t0 = q * 4
            for kk in range(4):
                t = t0 + kk

                @pl.when(t + 1 < CHUNKS)
                def _nx():
                    sg(t + 1, (kk + 1) % 4, (kk + 1) % 2)

                @pl.when(t + 2 < CHUNKS)
                def _pf():
                    si(t + 2, (kk + 2) % 4)

                fin(t, kk, kk % 2)
            return _

        lax.fori_loop(0, CHUNKS // 4, step, 0)
        plsc.subcore_barrier()

        @pl.when(c == 0)
        def _w0():
            pltpu.sync_copy(agg_sh.at[pl.ds(s * ZR, ZR)],
                            agg0.at[pl.ds(s * ZR, ZR)])

        @pl.when(c == 1)
        def _w1():
            pltpu.sync_copy(agg_sh.at[pl.ds(s * ZR, ZR)],
                            agg1.at[pl.ds(s * ZR, ZR)])


def _sc_msg(hx, wf, rowp, colp, zrows):
    return pl.kernel(
        _msg_body,
        out_type=(jax.ShapeDtypeStruct((NP, H), jnp.float32),
                  jax.ShapeDtypeStruct((NP, H), jnp.float32)),
        mesh=_MESH,
        scratch_types=[
            [pltpu.VMEM((CH,), jnp.int32) for _ in range(4)],
            [pltpu.VMEM((CH,), jnp.int32) for _ in range(4)],
            [pltpu.VMEM((CH, H), jnp.float32) for _ in range(2)],
            [pltpu.VMEM((CH, H), jnp.float32) for _ in range(2)],
            [pltpu.SemaphoreType.DMA for _ in range(4)],
            [pltpu.SemaphoreType.DMA for _ in range(4)],
            [pltpu.SemaphoreType.DMA for _ in range(2)],
            [pltpu.SemaphoreType.DMA for _ in range(2)],
            pltpu.VMEM_SHARED((NP, H), jnp.float32),
        ],
        cost_estimate=pl.CostEstimate(
            flops=2 * EP * H,
            bytes_accessed=EP * (8 * H + 16) + 3 * NC * NP * H * 4,
            transcendentals=0,
        ),
    )(hx, wf, rowp, colp, zrows)


def _upd_body(a0_ref, a1_ref, h_ref, l2_ref, b2_ref, l1n_ref,
              hn_ref, hxn_ref):
    agg = a0_ref[...] + a1_ref[...]
    hn = h_ref[...] + jnp.dot(agg, l2_ref[...],
                              preferred_element_type=jnp.float32) + b2_ref[...]
    hn_ref[...] = hn
    hxn_ref[...] = jnp.dot(hn, l1n_ref[...],
                           preferred_element_type=jnp.float32)


def _tc_update(agg0, agg1, h, lin2_i, lin2_b_i, lin1_next):
    return pl.pallas_call(
        _upd_body,
        grid=(NNB,),
        in_specs=[
            pl.BlockSpec((BN, H), lambda i: (i, 0)),
            pl.BlockSpec((BN, H), lambda i: (i, 0)),
            pl.BlockSpec((BN, H), lambda i: (i, 0)),
            pl.BlockSpec((H, H), lambda i: (0, 0)),
            pl.BlockSpec((1, H), lambda i: (0, 0)),
            pl.BlockSpec((H, H), lambda i: (0, 0)),
        ],
        out_specs=[pl.BlockSpec((BN, H), lambda i: (i, 0)),
                   pl.BlockSpec((BN, H), lambda i: (i, 0))],
        out_shape=[jax.ShapeDtypeStruct((N, H), jnp.float32),
                   jax.ShapeDtypeStruct((N, H), jnp.float32)],
    )(agg0, agg1, h, lin2_i, lin2_b_i[None, :], lin1_next)


def _pool_body(batch_ref, h_ref, e3_ref, w1a_ref, w1b_ref, b1_ref,
               w2_ref, b2_ref, w3_ref, b3_ref, out_ref, acc_ref, cnt_ref):
    i = pl.program_id(0)
    nb = pl.num_programs(0)

    @pl.when(i == 0)
    def _init():
        acc_ref[...] = jnp.zeros_like(acc_ref)
        cnt_ref[...] = jnp.zeros_like(cnt_ref)

    b = batch_ref[0, 0, :]
    gids = lax.broadcasted_iota(jnp.int32, (NGRAPHS, BN), 0)
    onehot = (b[None, :] == gids).astype(jnp.float32)
    acc_ref[...] += jnp.dot(onehot, h_ref[...],
                            preferred_element_type=jnp.float32)
    cnt_ref[...] += jnp.sum(onehot, axis=1)[None, :]

    @pl.when(i == nb - 1)
    def _final():
        counts = jnp.maximum(cnt_ref[0, :], 1.0)
        scale = 1.0 / (counts * jnp.sqrt(counts))
        pooled = acc_ref[...] * scale[:, None]
        o1 = _silu(pooled @ w1a_ref[...] + e3_ref[...] @ w1b_ref[...]
                   + b1_ref[...])
        o2 = _silu(o1 @ w2_ref[...] + b2_ref[...])
        out_ref[...] = o2 @ w3_ref[...] + b3_ref[...]


def _tc_pool(h, batch, e3_row, out1_W, out1_b, out2_W, out2_b,
             out3_W, out3_b):
    batch3 = batch.reshape(NNB, 1, BN)
    return pl.pallas_call(
        _pool_body,
        grid=(NNB,),
        in_specs=[
            pl.BlockSpec((1, 1, BN), lambda i: (i, 0, 0)),
            pl.BlockSpec((BN, H), lambda i: (i, 0)),
            pl.BlockSpec((1, H), lambda i: (0, 0)),
            pl.BlockSpec((H, H), lambda i: (0, 0)),
            pl.BlockSpec((H, H), lambda i: (0, 0)),
            pl.BlockSpec((1, H), lambda i: (0, 0)),
            pl.BlockSpec((H, H // 2), lambda i: (0, 0)),
            pl.BlockSpec((1, H // 2), lambda i: (0, 0)),
            pl.BlockSpec((H // 2, 1), lambda i: (0, 0)),
            pl.BlockSpec((1, 1), lambda i: (0, 0)),
        ],
        out_specs=pl.BlockSpec((NGRAPHS, 1), lambda i: (0, 0)),
        out_shape=jax.ShapeDtypeStruct((NGRAPHS, 1), jnp.float32),
        scratch_shapes=[
            pltpu.VMEM((NGRAPHS, H), jnp.float32),
            pltpu.VMEM((1, NGRAPHS), jnp.float32),
        ],
    )(batch3, h, e3_row, out1_W[:H], out1_W[H:], out1_b[None, :],
      out2_W, out2_b[None, :], out3_W, out3_b[None, :])


def kernel(x, pos, edge_index, batch, e3_idx, W_emb, b_emb, lin1_W, lin2_W,
           lin2_b, fn1_W, fn1_b, fn2_W, fn2_b, e3_table, out1_W, out1_b,
           out2_W, out2_b, out3_W, out3_b):
    row = edge_index[0]
    col = edge_index[1]
    pad_i = jnp.arange(PAD, dtype=jnp.int32)
    rowp = jnp.concatenate([row, pad_i % N])
    colp = jnp.concatenate([col, N + (pad_i % (NP - N))])
    px = jnp.pad(pos[:, 0], (0, NPAD - N))
    py = jnp.pad(pos[:, 1], (0, NPAD - N))
    pz = jnp.pad(pos[:, 2], (0, NPAD - N))
    fn1p = jnp.pad(fn1_W, ((0, 0), (0, NGP - NG), (0, 0))).astype(jnp.bfloat16)
    fn2b16 = fn2_W.astype(jnp.bfloat16)
    zrows = jnp.zeros((ZR, H), jnp.float32)

    ew2b = _sc_geo(px, py, pz, rowp, colp)
    eattr = _tc_attr(ew2b)
    wfs = [_tc_filter(eattr, fn1p[i], fn1_b[i], fn2b16[i], fn2_b[i])
           for i in range(NUM_INTER)]

    h, hx = _tc_emb(x, W_emb, b_emb, lin1_W[0])
    for i in range(NUM_INTER):
        agg0, agg1 = _sc_msg(hx, wfs[i], rowp, colp, zrows)
        lin1_next = lin1_W[(i + 1) % NUM_INTER]
        h, hx = _tc_update(agg0, agg1, h, lin2_W[i], lin2_b[i], lin1_next)

    e3_row = e3_table[e3_idx][None, :]
    return _tc_pool(h, batch, e3_row, out1_W, out1_b, out2_W, out2_b,
                    out3_W, out3_b)


# speedup vs baseline: 1.1811x; 1.1397x over previous
import jax
import jax.numpy as jnp
from jax import lax
from jax.experimental import pallas as pl
from jax.experimental.pallas import tpu as pltpu
from jax.experimental.pallas import tpu_sc as plsc

N = 10000
E = 640000
NODE_DIM = 28
H = 128
NG = 50
NGP = 64
NGRAPHS = 16
NUM_INTER = 4
CUTOFF = 10.0

NC, NS = 2, 16
NW = NC * NS
CH = 64
CHUNKS = 320
EW = CH * CHUNKS
EP = NW * EW
PAD = EP - E
GCH = 128
NP = 10240
ZR = NP // NS
BE = 1024
NEB = EP // BE
BN = 1000
NNB = N // BN
BU = 80
NUB = N // BU

_DELTA = CUTOFF / (NG - 1)
_COEFF = -0.5 / (_DELTA * _DELTA)

_MESH = plsc.VectorSubcoreMesh(core_axis_name="c", subcore_axis_name="s",
                               num_cores=NC, num_subcores=NS)


def _silu(v):
    return v * jax.nn.sigmoid(v)


NPAD = 10240


def _geo_body(px, py, pz, rowp, colp, ew2b,
              pxv, pyv, pzv, rvs, cvs, obs, rss, css, oss):
    wid = lax.axis_index("s") * NC + lax.axis_index("c")
    base = wid * EW
    pltpu.sync_copy(px, pxv)
    pltpu.sync_copy(py, pyv)
    pltpu.sync_copy(pz, pzv)

    def si(t, k):
        pltpu.async_copy(rowp.at[pl.ds(base + t * GCH, GCH)], rvs[k], rss[k])
        pltpu.async_copy(colp.at[pl.ds(base + t * GCH, GCH)], cvs[k], css[k])

    def fin(t, k):
        pltpu.make_async_copy(rowp.at[pl.ds(base + t * GCH, GCH)],
                              rvs[k], rss[k]).wait()
        pltpu.make_async_copy(colp.at[pl.ds(base + t * GCH, GCH)],
                              cvs[k], css[k]).wait()

        @pl.when(t >= 2)
        def _wo():
            pltpu.make_async_copy(
                obs[k], ew2b.at[:, pl.ds(base + (t - 2) * GCH, GCH)],
                oss[k]).wait()

        for g in range(GCH // 16):
            sl = pl.ds(g * 16, 16)
            ri = rvs[k][sl]
            ci = cvs[k][sl]
            dx = plsc.load_gather(pxv, [ri]) - plsc.load_gather(pxv, [ci])
            dy = plsc.load_gather(pyv, [ri]) - plsc.load_gather(pyv, [ci])
            dz = plsc.load_gather(pzv, [ri]) - plsc.load_gather(pzv, [ci])
            e2 = dx * dx + dy * dy + dz * dz
            for r in range(8):
                obs[k][r, sl] = e2
        pltpu.async_copy(obs[k], ew2b.at[:, pl.ds(base + t * GCH, GCH)],
                         oss[k])

    si(0, 0)
    si(1, 1)

    def step(u, _):
        t0 = u * 2
        fin(t0, 0)

        @pl.when(t0 + 2 < EW // GCH)
        def _p0():
            si(t0 + 2, 0)

        fin(t0 + 1, 1)

        @pl.when(t0 + 3 < EW // GCH)
        def _p1():
            si(t0 + 3, 1)

        return _

    lax.fori_loop(0, EW // GCH // 2, step, 0)
    nt = EW // GCH
    pltpu.make_async_copy(obs[0], ew2b.at[:, pl.ds(base + (nt - 2) * GCH,
                                                   GCH)], oss[0]).wait()
    pltpu.make_async_copy(obs[1], ew2b.at[:, pl.ds(base + (nt - 1) * GCH,
                                                   GCH)], oss[1]).wait()


def _sc_geo(px, py, pz, rowp, colp):
    return pl.kernel(
        _geo_body,
        out_type=jax.ShapeDtypeStruct((8, EP), jnp.float32),
        mesh=_MESH,
        scratch_types=[
            pltpu.VMEM((NPAD,), jnp.float32),
            pltpu.VMEM((NPAD,), jnp.float32),
            pltpu.VMEM((NPAD,), jnp.float32),
            [pltpu.VMEM((GCH,), jnp.int32) for _ in range(2)],
            [pltpu.VMEM((GCH,), jnp.int32) for _ in range(2)],
            [pltpu.VMEM((8, GCH), jnp.float32) for _ in range(2)],
            [pltpu.SemaphoreType.DMA for _ in range(2)],
            [pltpu.SemaphoreType.DMA for _ in range(2)],
            [pltpu.SemaphoreType.DMA for _ in range(2)],
        ],
        compiler_params=pltpu.CompilerParams(needs_layout_passes=False),
    )(px, py, pz, rowp, colp)


def _filter_body(ew2_ref, w1_ref, b1_ref, w2_ref, b2_ref, out_ref):
    ew2 = jnp.transpose(ew2_ref[...])[:, 0:1]
    ew = jnp.sqrt(ew2)
    ki = lax.broadcasted_iota(jnp.int32, (BE, NGP), 1)
    dd = ew - ki.astype(jnp.float32) * _DELTA
    mask = ki < NG
    a = jnp.where(mask, jnp.exp(_COEFF * dd * dd), 0.0).astype(jnp.bfloat16)
    z = jnp.dot(a, w1_ref[...], preferred_element_type=jnp.float32)
    z = _silu(z + b1_ref[...])
    out_ref[...] = jnp.dot(z.astype(jnp.bfloat16), w2_ref[...],
                           preferred_element_type=jnp.float32) + b2_ref[...]


def _tc_filter(ew2b, fn1p_l, fn1_b_l, fn2_W_l, fn2_b_l):
    return pl.pallas_call(
        _filter_body,
        grid=(NEB,),
        in_specs=[
            pl.BlockSpec((8, BE), lambda e: (0, e)),
            pl.BlockSpec((NGP, H), lambda e: (0, 0)),
            pl.BlockSpec((1, H), lambda e: (0, 0)),
            pl.BlockSpec((H, H), lambda e: (0, 0)),
            pl.BlockSpec((1, H), lambda e: (0, 0)),
        ],
        out_specs=pl.BlockSpec((BE, H), lambda e: (e, 0)),
        out_shape=jax.ShapeDtypeStruct((EP, H), jnp.float32),
    )(ew2b, fn1p_l, fn1_b_l[None, :], fn2_W_l, fn2_b_l[None, :])


def _emb_body(x_ref, we_ref, be_ref, l1_ref, h_ref, hx_ref):
    h = jnp.dot(x_ref[...], we_ref[...],
                preferred_element_type=jnp.float32) + be_ref[...]
    h_ref[...] = h
    hx_ref[...] = jnp.dot(h, l1_ref[...], preferred_element_type=jnp.float32)


def _tc_emb(x, W_emb, b_emb, lin1_0):
    return pl.pallas_call(
        _emb_body,
        grid=(NNB,),
        in_specs=[
            pl.BlockSpec((BN, NODE_DIM), lambda i: (i, 0)),
            pl.BlockSpec((NODE_DIM, H), lambda i: (0, 0)),
            pl.BlockSpec((1, H), lambda i: (0, 0)),
            pl.BlockSpec((H, H), lambda i: (0, 0)),
        ],
        out_specs=[pl.BlockSpec((BN, H), lambda i: (i, 0)),
                   pl.BlockSpec((BN, H), lambda i: (i, 0))],
        out_shape=[jax.ShapeDtypeStruct((N, H), jnp.float32),
                   jax.ShapeDtypeStruct((N, H), jnp.float32)],
    )(x, W_emb, b_emb[None, :], lin1_0)


def _msg_body(hx, wf, rowp, colp, zrows, agg0, agg1,
             rows, cols, ghxs, wfvs, rss, css, gss, wss, agg_sh):
        c = lax.axis_index("c")
        s = lax.axis_index("s")
        wid = s * NC + c
        base = wid * EW

        pltpu.sync_copy(zrows, agg_sh.at[pl.ds(s * ZR, ZR)])
        plsc.subcore_barrier()

        def si(t, k):
            pltpu.async_copy(rowp.at[pl.ds(base + t * CH, CH)], rows[k],
                             rss[k])
            pltpu.async_copy(colp.at[pl.ds(base + t * CH, CH)], cols[k],
                             css[k])

        def sg(t, k, b):
            pltpu.make_async_copy(rowp.at[pl.ds(base + t * CH, CH)],
                                  rows[k], rss[k]).wait()
            pltpu.async_copy(hx.at[rows[k]], ghxs[b], gss[b])
            pltpu.async_copy(wf.at[pl.ds(base + t * CH, CH)],
                             wfvs[b], wss[b])

        def fin(t, k, b):
            ghx, wfv = ghxs[b], wfvs[b]
            pltpu.make_async_copy(hx.at[rows[k]], ghx, gss[b]).wait()
            pltpu.make_async_copy(wf.at[pl.ds(base + t * CH, CH)],
                                  wfv, wss[b]).wait()
            pltpu.make_async_copy(colp.at[pl.ds(base + t * CH, CH)],
                                  cols[k], css[k]).wait()

            @plsc.parallel_loop(0, CH, 1, unroll=4)
            def _mul(r):
                for j in range(H // 16):
                    sl = pl.ds(j * 16, 16)
                    ghx[r, sl] = ghx[r, sl] * wfv[r, sl]

            pltpu.sync_copy(ghx, agg_sh.at[cols[k]], add=True)

        si(0, 0)
        si(1, 1)
        sg(0, 0, 0)

        def step(q, _):
            t0 = q * 4
            for kk in range(4):
                t = t0 + kk

                @pl.when(t + 1 < CHUNKS)
                def _nx():
                    sg(t + 1, (kk + 1) % 4, (kk + 1) % 2)

                @pl.when(t + 2 < CHUNKS)
                def _pf():
                    si(t + 2, (kk + 2) % 4)

                fin(t, kk, kk % 2)
            return _

        lax.fori_loop(0, CHUNKS // 4, step, 0)
        plsc.subcore_barrier()

        @pl.when(c == 0)
        def _w0():
            pltpu.sync_copy(agg_sh.at[pl.ds(s * ZR, ZR)],
                            agg0.at[pl.ds(s * ZR, ZR)])

        @pl.when(c == 1)
        def _w1():
            pltpu.sync_copy(agg_sh.at[pl.ds(s * ZR, ZR)],
                            agg1.at[pl.ds(s * ZR, ZR)])


def _sc_msg(hx, wf, rowp, colp, zrows):
    return pl.kernel(
        _msg_body,
        out_type=(jax.ShapeDtypeStruct((NP, H), jnp.float32),
                  jax.ShapeDtypeStruct((NP, H), jnp.float32)),
        mesh=_MESH,
        scratch_types=[
            [pltpu.VMEM((CH,), jnp.int32) for _ in range(4)],
            [pltpu.VMEM((CH,), jnp.int32) for _ in range(4)],
            [pltpu.VMEM((CH, H), jnp.float32) for _ in range(2)],
            [pltpu.VMEM((CH, H), jnp.float32) for _ in range(2)],
            [pltpu.SemaphoreType.DMA for _ in range(4)],
            [pltpu.SemaphoreType.DMA for _ in range(4)],
            [pltpu.SemaphoreType.DMA for _ in range(2)],
            [pltpu.SemaphoreType.DMA for _ in range(2)],
            pltpu.VMEM_SHARED((NP, H), jnp.float32),
        ],
        cost_estimate=pl.CostEstimate(
            flops=2 * EP * H,
            bytes_accessed=EP * (8 * H + 16) + 3 * NC * NP * H * 4,
            transcendentals=0,
        ),
    )(hx, wf, rowp, colp, zrows)


def _upd_body(a0_ref, a1_ref, h_ref, l2_ref, b2_ref, l1n_ref,
              hn_ref, hxn_ref):
    agg = a0_ref[...] + a1_ref[...]
    hn = h_ref[...] + jnp.dot(agg, l2_ref[...],
                              preferred_element_type=jnp.float32) + b2_ref[...]
    hn_ref[...] = hn
    hxn_ref[...] = jnp.dot(hn, l1n_ref[...],
                           preferred_element_type=jnp.float32)


def _tc_update(agg0, agg1, h, lin2_i, lin2_b_i, lin1_next):
    return pl.pallas_call(
        _upd_body,
        grid=(NNB,),
        in_specs=[
            pl.BlockSpec((BN, H), lambda i: (i, 0)),
            pl.BlockSpec((BN, H), lambda i: (i, 0)),
            pl.BlockSpec((BN, H), lambda i: (i, 0)),
            pl.BlockSpec((H, H), lambda i: (0, 0)),
            pl.BlockSpec((1, H), lambda i: (0, 0)),
            pl.BlockSpec((H, H), lambda i: (0, 0)),
        ],
        out_specs=[pl.BlockSpec((BN, H), lambda i: (i, 0)),
                   pl.BlockSpec((BN, H), lambda i: (i, 0))],
        out_shape=[jax.ShapeDtypeStruct((N, H), jnp.float32),
                   jax.ShapeDtypeStruct((N, H), jnp.float32)],
    )(agg0, agg1, h, lin2_i, lin2_b_i[None, :], lin1_next)


def _pool_body(batch_ref, h_ref, e3_ref, w1a_ref, w1b_ref, b1_ref,
               w2_ref, b2_ref, w3_ref, b3_ref, out_ref, acc_ref, cnt_ref):
    i = pl.program_id(0)
    nb = pl.num_programs(0)

    @pl.when(i == 0)
    def _init():
        acc_ref[...] = jnp.zeros_like(acc_ref)
        cnt_ref[...] = jnp.zeros_like(cnt_ref)

    b = batch_ref[0, 0, :]
    gids = lax.broadcasted_iota(jnp.int32, (NGRAPHS, BN), 0)
    onehot = (b[None, :] == gids).astype(jnp.float32)
    acc_ref[...] += jnp.dot(onehot, h_ref[...],
                            preferred_element_type=jnp.float32)
    cnt_ref[...] += jnp.sum(onehot, axis=1)[None, :]

    @pl.when(i == nb - 1)
    def _final():
        counts = jnp.maximum(cnt_ref[0, :], 1.0)
        scale = 1.0 / (counts * jnp.sqrt(counts))
        pooled = acc_ref[...] * scale[:, None]
        o1 = _silu(pooled @ w1a_ref[...] + e3_ref[...] @ w1b_ref[...]
                   + b1_ref[...])
        o2 = _silu(o1 @ w2_ref[...] + b2_ref[...])
        out_ref[...] = o2 @ w3_ref[...] + b3_ref[...]


def _tc_pool(h, batch, e3_row, out1_W, out1_b, out2_W, out2_b,
             out3_W, out3_b):
    batch3 = batch.reshape(NNB, 1, BN)
    return pl.pallas_call(
        _pool_body,
        grid=(NNB,),
        in_specs=[
            pl.BlockSpec((1, 1, BN), lambda i: (i, 0, 0)),
            pl.BlockSpec((BN, H), lambda i: (i, 0)),
            pl.BlockSpec((1, H), lambda i: (0, 0)),
            pl.BlockSpec((H, H), lambda i: (0, 0)),
            pl.BlockSpec((H, H), lambda i: (0, 0)),
            pl.BlockSpec((1, H), lambda i: (0, 0)),
            pl.BlockSpec((H, H // 2), lambda i: (0, 0)),
            pl.BlockSpec((1, H // 2), lambda i: (0, 0)),
            pl.BlockSpec((H // 2, 1), lambda i: (0, 0)),
            pl.BlockSpec((1, 1), lambda i: (0, 0)),
        ],
        out_specs=pl.BlockSpec((NGRAPHS, 1), lambda i: (0, 0)),
        out_shape=jax.ShapeDtypeStruct((NGRAPHS, 1), jnp.float32),
        scratch_shapes=[
            pltpu.VMEM((NGRAPHS, H), jnp.float32),
            pltpu.VMEM((1, NGRAPHS), jnp.float32),
        ],
    )(batch3, h, e3_row, out1_W[:H], out1_W[H:], out1_b[None, :],
      out2_W, out2_b[None, :], out3_W, out3_b[None, :])


def kernel(x, pos, edge_index, batch, e3_idx, W_emb, b_emb, lin1_W, lin2_W,
           lin2_b, fn1_W, fn1_b, fn2_W, fn2_b, e3_table, out1_W, out1_b,
           out2_W, out2_b, out3_W, out3_b):
    row = edge_index[0]
    col = edge_index[1]
    pad_i = jnp.arange(PAD, dtype=jnp.int32)
    rowp = jnp.concatenate([row, pad_i % N])
    colp = jnp.concatenate([col, N + (pad_i % (NP - N))])
    px = jnp.pad(pos[:, 0], (0, NPAD - N))
    py = jnp.pad(pos[:, 1], (0, NPAD - N))
    pz = jnp.pad(pos[:, 2], (0, NPAD - N))
    fn1p = jnp.pad(fn1_W, ((0, 0), (0, NGP - NG), (0, 0))).astype(jnp.bfloat16)
    fn2b16 = fn2_W.astype(jnp.bfloat16)
    zrows = jnp.zeros((ZR, H), jnp.float32)

    ew2b = _sc_geo(px, py, pz, rowp, colp)
    wfs = [_tc_filter(ew2b, fn1p[i], fn1_b[i], fn2b16[i], fn2_b[i])
           for i in range(NUM_INTER)]

    h, hx = _tc_emb(x, W_emb, b_emb, lin1_W[0])
    for i in range(NUM_INTER):
        agg0, agg1 = _sc_msg(hx, wfs[i], rowp, colp, zrows)
        lin1_next = lin1_W[(i + 1) % NUM_INTER]
        h, hx = _tc_update(agg0, agg1, h, lin2_W[i], lin2_b[i], lin1_next)

    e3_row = e3_table[e3_idx][None, :]
    return _tc_pool(h, batch, e3_row, out1_W, out1_b, out2_W, out2_b,
                    out3_W, out3_b)


# speedup vs baseline: 1.5481x; 1.3107x over previous
import jax
import jax.numpy as jnp
from jax import lax
from jax.experimental import pallas as pl
from jax.experimental.pallas import tpu as pltpu
from jax.experimental.pallas import tpu_sc as plsc

N = 10000
E = 640000
NODE_DIM = 28
H = 128
NG = 50
NGP = 64
NGRAPHS = 16
NUM_INTER = 4
CUTOFF = 10.0

NC, NS = 2, 16
NW = NC * NS
CH = 64
CHUNKS = 320
EW = CH * CHUNKS
EP = NW * EW
PAD = EP - E
GCH = 128
NP = 10240
ZR = NP // NS
BE = 2048
NEB = EP // BE
BN = 1000
NNB = N // BN
BU = 80
NUB = N // BU

_DELTA = CUTOFF / (NG - 1)
_COEFF = -0.5 / (_DELTA * _DELTA)

_MESH = plsc.VectorSubcoreMesh(core_axis_name="c", subcore_axis_name="s",
                               num_cores=NC, num_subcores=NS)


def _silu(v):
    return v * jax.nn.sigmoid(v)


NPAD = 10240


def _geo_body(px, py, pz, rowp, colp, ew2b,
              pxv, pyv, pzv, rvs, cvs, obs, rss, css, oss):
    wid = lax.axis_index("s") * NC + lax.axis_index("c")
    base = wid * EW
    pltpu.sync_copy(px, pxv)
    pltpu.sync_copy(py, pyv)
    pltpu.sync_copy(pz, pzv)

    def si(t, k):
        pltpu.async_copy(rowp.at[pl.ds(base + t * GCH, GCH)], rvs[k], rss[k])
        pltpu.async_copy(colp.at[pl.ds(base + t * GCH, GCH)], cvs[k], css[k])

    def fin(t, k):
        pltpu.make_async_copy(rowp.at[pl.ds(base + t * GCH, GCH)],
                              rvs[k], rss[k]).wait()
        pltpu.make_async_copy(colp.at[pl.ds(base + t * GCH, GCH)],
                              cvs[k], css[k]).wait()

        @pl.when(t >= 2)
        def _wo():
            pltpu.make_async_copy(
                obs[k], ew2b.at[:, pl.ds(base + (t - 2) * GCH, GCH)],
                oss[k]).wait()

        for g in range(GCH // 16):
            sl = pl.ds(g * 16, 16)
            ri = rvs[k][sl]
            ci = cvs[k][sl]
            dx = plsc.load_gather(pxv, [ri]) - plsc.load_gather(pxv, [ci])
            dy = plsc.load_gather(pyv, [ri]) - plsc.load_gather(pyv, [ci])
            dz = plsc.load_gather(pzv, [ri]) - plsc.load_gather(pzv, [ci])
            e2 = dx * dx + dy * dy + dz * dz
            for r in range(8):
                obs[k][r, sl] = e2
        pltpu.async_copy(obs[k], ew2b.at[:, pl.ds(base + t * GCH, GCH)],
                         oss[k])

    si(0, 0)
    si(1, 1)

    def step(u, _):
        t0 = u * 2
        fin(t0, 0)

        @pl.when(t0 + 2 < EW // GCH)
        def _p0():
            si(t0 + 2, 0)

        fin(t0 + 1, 1)

        @pl.when(t0 + 3 < EW // GCH)
        def _p1():
            si(t0 + 3, 1)

        return _

    lax.fori_loop(0, EW // GCH // 2, step, 0)
    nt = EW // GCH
    pltpu.make_async_copy(obs[0], ew2b.at[:, pl.ds(base + (nt - 2) * GCH,
                                                   GCH)], oss[0]).wait()
    pltpu.make_async_copy(obs[1], ew2b.at[:, pl.ds(base + (nt - 1) * GCH,
                                                   GCH)], oss[1]).wait()


def _sc_geo(px, py, pz, rowp, colp):
    return pl.kernel(
        _geo_body,
        out_type=jax.ShapeDtypeStruct((8, EP), jnp.float32),
        mesh=_MESH,
        scratch_types=[
            pltpu.VMEM((NPAD,), jnp.float32),
            pltpu.VMEM((NPAD,), jnp.float32),
            pltpu.VMEM((NPAD,), jnp.float32),
            [pltpu.VMEM((GCH,), jnp.int32) for _ in range(2)],
            [pltpu.VMEM((GCH,), jnp.int32) for _ in range(2)],
            [pltpu.VMEM((8, GCH), jnp.float32) for _ in range(2)],
            [pltpu.SemaphoreType.DMA for _ in range(2)],
            [pltpu.SemaphoreType.DMA for _ in range(2)],
            [pltpu.SemaphoreType.DMA for _ in range(2)],
        ],
        compiler_params=pltpu.CompilerParams(needs_layout_passes=False),
    )(px, py, pz, rowp, colp)


def _filter_body(ew2_ref, w1_ref, b1_ref, w2_ref, b2_ref, out_ref):
    ew2 = jnp.transpose(ew2_ref[...])[:, 0:1]
    ew = jnp.sqrt(ew2)
    ki = lax.broadcasted_iota(jnp.int32, (BE, NGP), 1)
    dd = ew - ki.astype(jnp.float32) * _DELTA
    mask = ki < NG
    a = jnp.where(mask, jnp.exp(_COEFF * dd * dd), 0.0).astype(jnp.bfloat16)
    z = jnp.dot(a, w1_ref[...], preferred_element_type=jnp.float32)
    z = _silu(z + b1_ref[...])
    out_ref[...] = jnp.dot(z.astype(jnp.bfloat16), w2_ref[...],
                           preferred_element_type=jnp.float32) + b2_ref[...]


def _tc_filter(ew2b, fn1p_l, fn1_b_l, fn2_W_l, fn2_b_l):
    return pl.pallas_call(
        _filter_body,
        grid=(NEB,),
        in_specs=[
            pl.BlockSpec((8, BE), lambda e: (0, e)),
            pl.BlockSpec((NGP, H), lambda e: (0, 0)),
            pl.BlockSpec((1, H), lambda e: (0, 0)),
            pl.BlockSpec((H, H), lambda e: (0, 0)),
            pl.BlockSpec((1, H), lambda e: (0, 0)),
        ],
        out_specs=pl.BlockSpec((BE, H), lambda e: (e, 0)),
        out_shape=jax.ShapeDtypeStruct((EP, H), jnp.float32),
    )(ew2b, fn1p_l, fn1_b_l[None, :], fn2_W_l, fn2_b_l[None, :])


def _emb_body(x_ref, we_ref, be_ref, l1_ref, h_ref, hx_ref):
    h = jnp.dot(x_ref[...], we_ref[...],
                preferred_element_type=jnp.float32) + be_ref[...]
    h_ref[...] = h
    hx_ref[...] = jnp.dot(h, l1_ref[...], preferred_element_type=jnp.float32)


def _tc_emb(x, W_emb, b_emb, lin1_0):
    return pl.pallas_call(
        _emb_body,
        grid=(NNB,),
        in_specs=[
            pl.BlockSpec((BN, NODE_DIM), lambda i: (i, 0)),
            pl.BlockSpec((NODE_DIM, H), lambda i: (0, 0)),
            pl.BlockSpec((1, H), lambda i: (0, 0)),
            pl.BlockSpec((H, H), lambda i: (0, 0)),
        ],
        out_specs=[pl.BlockSpec((BN, H), lambda i: (i, 0)),
                   pl.BlockSpec((BN, H), lambda i: (i, 0))],
        out_shape=[jax.ShapeDtypeStruct((N, H), jnp.float32),
                   jax.ShapeDtypeStruct((N, H), jnp.float32)],
    )(x, W_emb, b_emb[None, :], lin1_0)


def _msg_body(hx, wf, rowp, colp, zrows, agg0, agg1,
             rows, cols, ghxs, wfvs, rss, css, gss, wss, agg_sh):
        c = lax.axis_index("c")
        s = lax.axis_index("s")
        wid = s * NC + c
        base = wid * EW

        pltpu.sync_copy(zrows, agg_sh.at[pl.ds(s * ZR, ZR)])
        plsc.subcore_barrier()

        def si(t, k):
            pltpu.async_copy(rowp.at[pl.ds(base + t * CH, CH)], rows[k],
                             rss[k])
            pltpu.async_copy(colp.at[pl.ds(base + t * CH, CH)], cols[k],
                             css[k])

        def sg(t, k, b):
            pltpu.make_async_copy(rowp.at[pl.ds(base + t * CH, CH)],
                                  rows[k], rss[k]).wait()
            pltpu.async_copy(hx.at[rows[k]], ghxs[b], gss[b])
            pltpu.async_copy(wf.at[pl.ds(base + t * CH, CH)],
                             wfvs[b], wss[b])

        def fin(t, k, b):
            ghx, wfv = ghxs[b], wfvs[b]
            pltpu.make_async_copy(hx.at[rows[k]], ghx, gss[b]).wait()
            pltpu.make_async_copy(wf.at[pl.ds(base + t * CH, CH)],
                                  wfv, wss[b]).wait()
            pltpu.make_async_copy(colp.at[pl.ds(base + t * CH, CH)],
                                  cols[k], css[k]).wait()

            @plsc.parallel_loop(0, CH, 1, unroll=4)
            def _mul(r):
                for j in range(H // 16):
                    sl = pl.ds(j * 16, 16)
                    ghx[r, sl] = ghx[r, sl] * wfv[r, sl]

            pltpu.sync_copy(ghx, agg_sh.at[cols[k]], add=True)

        si(0, 0)
        si(1, 1)
        sg(0, 0, 0)

        def step(q, _):
            t0 = q * 4
            for kk in range(4):
                t = t0 + kk

                @pl.when(t + 1 < CHUNKS)
                def _nx():
                    sg(t + 1, (kk + 1) % 4, (kk + 1) % 2)

                @pl.when(t + 2 < CHUNKS)
                def _pf():
                    si(t + 2, (kk + 2) % 4)

                fin(t, kk, kk % 2)
            return _

        lax.fori_loop(0, CHUNKS // 4, step, 0)
        plsc.subcore_barrier()

        @pl.when(c == 0)
        def _w0():
            pltpu.sync_copy(agg_sh.at[pl.ds(s * ZR, ZR)],
                            agg0.at[pl.ds(s * ZR, ZR)])

        @pl.when(c == 1)
        def _w1():
            pltpu.sync_copy(agg_sh.at[pl.ds(s * ZR, ZR)],
                            agg1.at[pl.ds(s * ZR, ZR)])


def _sc_msg(hx, wf, rowp, colp, zrows):
    return pl.kernel(
        _msg_body,
        out_type=(jax.ShapeDtypeStruct((NP, H), jnp.float32),
                  jax.ShapeDtypeStruct((NP, H), jnp.float32)),
        mesh=_MESH,
        scratch_types=[
            [pltpu.VMEM((CH,), jnp.int32) for _ in range(4)],
            [pltpu.VMEM((CH,), jnp.int32) for _ in range(4)],
            [pltpu.VMEM((CH, H), jnp.float32) for _ in range(2)],
            [pltpu.VMEM((CH, H), jnp.float32) for _ in range(2)],
            [pltpu.SemaphoreType.DMA for _ in range(4)],
            [pltpu.SemaphoreType.DMA for _ in range(4)],
            [pltpu.SemaphoreType.DMA for _ in range(2)],
            [pltpu.SemaphoreType.DMA for _ in range(2)],
            pltpu.VMEM_SHARED((NP, H), jnp.float32),
        ],
        compiler_params=pltpu.CompilerParams(needs_layout_passes=False),
        cost_estimate=pl.CostEstimate(
            flops=2 * EP * H,
            bytes_accessed=EP * (8 * H + 16) + 3 * NC * NP * H * 4,
            transcendentals=0,
        ),
    )(hx, wf, rowp, colp, zrows)


def _upd_body(a0_ref, a1_ref, h_ref, l2_ref, b2_ref, l1n_ref,
              hn_ref, hxn_ref):
    agg = a0_ref[...] + a1_ref[...]
    hn = h_ref[...] + jnp.dot(agg, l2_ref[...],
                              preferred_element_type=jnp.float32) + b2_ref[...]
    hn_ref[...] = hn
    hxn_ref[...] = jnp.dot(hn, l1n_ref[...],
                           preferred_element_type=jnp.float32)


def _tc_update(agg0, agg1, h, lin2_i, lin2_b_i, lin1_next):
    return pl.pallas_call(
        _upd_body,
        grid=(NNB,),
        in_specs=[
            pl.BlockSpec((BN, H), lambda i: (i, 0)),
            pl.BlockSpec((BN, H), lambda i: (i, 0)),
            pl.BlockSpec((BN, H), lambda i: (i, 0)),
            pl.BlockSpec((H, H), lambda i: (0, 0)),
            pl.BlockSpec((1, H), lambda i: (0, 0)),
            pl.BlockSpec((H, H), lambda i: (0, 0)),
        ],
        out_specs=[pl.BlockSpec((BN, H), lambda i: (i, 0)),
                   pl.BlockSpec((BN, H), lambda i: (i, 0))],
        out_shape=[jax.ShapeDtypeStruct((N, H), jnp.float32),
                   jax.ShapeDtypeStruct((N, H), jnp.float32)],
    )(agg0, agg1, h, lin2_i, lin2_b_i[None, :], lin1_next)


def _pool_body(batch_ref, h_ref, e3_ref, w1a_ref, w1b_ref, b1_ref,
               w2_ref, b2_ref, w3_ref, b3_ref, out_ref, acc_ref, cnt_ref):
    i = pl.program_id(0)
    nb = pl.num_programs(0)

    @pl.when(i == 0)
    def _init():
        acc_ref[...] = jnp.zeros_like(acc_ref)
        cnt_ref[...] = jnp.zeros_like(cnt_ref)

    b = batch_ref[0, 0, :]
    gids = lax.broadcasted_iota(jnp.int32, (NGRAPHS, BN), 0)
    onehot = (b[None, :] == gids).astype(jnp.float32)
    acc_ref[...] += jnp.dot(onehot, h_ref[...],
                            preferred_element_type=jnp.float32)
    cnt_ref[...] += jnp.sum(onehot, axis=1)[None, :]

    @pl.when(i == nb - 1)
    def _final():
        counts = jnp.maximum(cnt_ref[0, :], 1.0)
        scale = 1.0 / (counts * jnp.sqrt(counts))
        pooled = acc_ref[...] * scale[:, None]
        o1 = _silu(pooled @ w1a_ref[...] + e3_ref[...] @ w1b_ref[...]
                   + b1_ref[...])
        o2 = _silu(o1 @ w2_ref[...] + b2_ref[...])
        out_ref[...] = o2 @ w3_ref[...] + b3_ref[...]


def _tc_pool(h, batch, e3_row, out1_W, out1_b, out2_W, out2_b,
             out3_W, out3_b):
    batch3 = batch.reshape(NNB, 1, BN)
    return pl.pallas_call(
        _pool_body,
        grid=(NNB,),
        in_specs=[
            pl.BlockSpec((1, 1, BN), lambda i: (i, 0, 0)),
            pl.BlockSpec((BN, H), lambda i: (i, 0)),
            pl.BlockSpec((1, H), lambda i: (0, 0)),
            pl.BlockSpec((H, H), lambda i: (0, 0)),
            pl.BlockSpec((H, H), lambda i: (0, 0)),
            pl.BlockSpec((1, H), lambda i: (0, 0)),
            pl.BlockSpec((H, H // 2), lambda i: (0, 0)),
            pl.BlockSpec((1, H // 2), lambda i: (0, 0)),
            pl.BlockSpec((H // 2, 1), lambda i: (0, 0)),
            pl.BlockSpec((1, 1), lambda i: (0, 0)),
        ],
        out_specs=pl.BlockSpec((NGRAPHS, 1), lambda i: (0, 0)),
        out_shape=jax.ShapeDtypeStruct((NGRAPHS, 1), jnp.float32),
        scratch_shapes=[
            pltpu.VMEM((NGRAPHS, H), jnp.float32),
            pltpu.VMEM((1, NGRAPHS), jnp.float32),
        ],
    )(batch3, h, e3_row, out1_W[:H], out1_W[H:], out1_b[None, :],
      out2_W, out2_b[None, :], out3_W, out3_b[None, :])


def kernel(x, pos, edge_index, batch, e3_idx, W_emb, b_emb, lin1_W, lin2_W,
           lin2_b, fn1_W, fn1_b, fn2_W, fn2_b, e3_table, out1_W, out1_b,
           out2_W, out2_b, out3_W, out3_b):
    row = edge_index[0]
    col = edge_index[1]
    pad_i = jnp.arange(PAD, dtype=jnp.int32)
    rowp = jnp.concatenate([row, pad_i % N])
    colp = jnp.concatenate([col, N + (pad_i % (NP - N))])
    px = jnp.pad(pos[:, 0], (0, NPAD - N))
    py = jnp.pad(pos[:, 1], (0, NPAD - N))
    pz = jnp.pad(pos[:, 2], (0, NPAD - N))
    fn1p = jnp.pad(fn1_W, ((0, 0), (0, NGP - NG), (0, 0))).astype(jnp.bfloat16)
    fn2b16 = fn2_W.astype(jnp.bfloat16)
    zrows = jnp.zeros((ZR, H), jnp.float32)

    ew2b = _sc_geo(px, py, pz, rowp, colp)
    wfs = [_tc_filter(ew2b, fn1p[i], fn1_b[i], fn2b16[i], fn2_b[i])
           for i in range(NUM_INTER)]

    h, hx = _tc_emb(x, W_emb, b_emb, lin1_W[0])
    for i in range(NUM_INTER):
        agg0, agg1 = _sc_msg(hx, wfs[i], rowp, colp, zrows)
        lin1_next = lin1_W[(i + 1) % NUM_INTER]
        h, hx = _tc_update(agg0, agg1, h, lin2_W[i], lin2_b[i], lin1_next)

    e3_row = e3_table[e3_idx][None, :]
    return _tc_pool(h, batch, e3_row, out1_W, out1_b, out2_W, out2_b,
                    out3_W, out3_b)


# speedup vs baseline: 1.5705x; 1.0145x over previous
import jax
import jax.numpy as jnp
from jax import lax
from jax.experimental import pallas as pl
from jax.experimental.pallas import tpu as pltpu
from jax.experimental.pallas import tpu_sc as plsc

N = 10000
E = 640000
NODE_DIM = 28
H = 128
NG = 50
NGP = 64
NGRAPHS = 16
NUM_INTER = 4
CUTOFF = 10.0

NC, NS = 2, 16
NW = NC * NS
CH = 64
CHUNKS = 320
EW = CH * CHUNKS
EP = NW * EW
PAD = EP - E
GCH = 128
NP = 10240
ZR = NP // NS
BE = 4096
NEB = EP // BE
BN = 1000
NNB = N // BN
BU = 80
NUB = N // BU

_DELTA = CUTOFF / (NG - 1)
_COEFF = -0.5 / (_DELTA * _DELTA)

_MESH = plsc.VectorSubcoreMesh(core_axis_name="c", subcore_axis_name="s",
                               num_cores=NC, num_subcores=NS)


def _silu(v):
    return v * jax.nn.sigmoid(v)


NPAD = 10240


def _geo_body(px, py, pz, rowp, colp, ew2b,
              pxv, pyv, pzv, rvs, cvs, obs, rss, css, oss):
    wid = lax.axis_index("s") * NC + lax.axis_index("c")
    base = wid * EW
    pltpu.sync_copy(px, pxv)
    pltpu.sync_copy(py, pyv)
    pltpu.sync_copy(pz, pzv)

    def si(t, k):
        pltpu.async_copy(rowp.at[pl.ds(base + t * GCH, GCH)], rvs[k], rss[k])
        pltpu.async_copy(colp.at[pl.ds(base + t * GCH, GCH)], cvs[k], css[k])

    def fin(t, k):
        pltpu.make_async_copy(rowp.at[pl.ds(base + t * GCH, GCH)],
                              rvs[k], rss[k]).wait()
        pltpu.make_async_copy(colp.at[pl.ds(base + t * GCH, GCH)],
                              cvs[k], css[k]).wait()

        @pl.when(t >= 2)
        def _wo():
            pltpu.make_async_copy(
                obs[k], ew2b.at[:, pl.ds(base + (t - 2) * GCH, GCH)],
                oss[k]).wait()

        for g in range(GCH // 16):
            sl = pl.ds(g * 16, 16)
            ri = rvs[k][sl]
            ci = cvs[k][sl]
            dx = plsc.load_gather(pxv, [ri]) - plsc.load_gather(pxv, [ci])
            dy = plsc.load_gather(pyv, [ri]) - plsc.load_gather(pyv, [ci])
            dz = plsc.load_gather(pzv, [ri]) - plsc.load_gather(pzv, [ci])
            e2 = dx * dx + dy * dy + dz * dz
            for r in range(8):
                obs[k][r, sl] = e2
        pltpu.async_copy(obs[k], ew2b.at[:, pl.ds(base + t * GCH, GCH)],
                         oss[k])

    si(0, 0)
    si(1, 1)

    def step(u, _):
        t0 = u * 2
        fin(t0, 0)

        @pl.when(t0 + 2 < EW // GCH)
        def _p0():
            si(t0 + 2, 0)

        fin(t0 + 1, 1)

        @pl.when(t0 + 3 < EW // GCH)
        def _p1():
            si(t0 + 3, 1)

        return _

    lax.fori_loop(0, EW // GCH // 2, step, 0)
    nt = EW // GCH
    pltpu.make_async_copy(obs[0], ew2b.at[:, pl.ds(base + (nt - 2) * GCH,
                                                   GCH)], oss[0]).wait()
    pltpu.make_async_copy(obs[1], ew2b.at[:, pl.ds(base + (nt - 1) * GCH,
                                                   GCH)], oss[1]).wait()


def _sc_geo(px, py, pz, rowp, colp):
    return pl.kernel(
        _geo_body,
        out_type=jax.ShapeDtypeStruct((8, EP), jnp.float32),
        mesh=_MESH,
        scratch_types=[
            pltpu.VMEM((NPAD,), jnp.float32),
            pltpu.VMEM((NPAD,), jnp.float32),
            pltpu.VMEM((NPAD,), jnp.float32),
            [pltpu.VMEM((GCH,), jnp.int32) for _ in range(2)],
            [pltpu.VMEM((GCH,), jnp.int32) for _ in range(2)],
            [pltpu.VMEM((8, GCH), jnp.float32) for _ in range(2)],
            [pltpu.SemaphoreType.DMA for _ in range(2)],
            [pltpu.SemaphoreType.DMA for _ in range(2)],
            [pltpu.SemaphoreType.DMA for _ in range(2)],
        ],
        compiler_params=pltpu.CompilerParams(needs_layout_passes=False),
    )(px, py, pz, rowp, colp)


def _filter_body(ew2_ref, w1_ref, b1_ref, w2_ref, b2_ref, out_ref):
    ew2 = jnp.transpose(ew2_ref[...])[:, 0:1]
    ew = jnp.sqrt(ew2)
    ki = lax.broadcasted_iota(jnp.int32, (BE, NGP), 1)
    dd = ew - ki.astype(jnp.float32) * _DELTA
    mask = ki < NG
    a = jnp.where(mask, jnp.exp(_COEFF * dd * dd), 0.0).astype(jnp.bfloat16)
    z = jnp.dot(a, w1_ref[...], preferred_element_type=jnp.float32)
    z = _silu(z + b1_ref[...])
    out_ref[...] = jnp.dot(z.astype(jnp.bfloat16), w2_ref[...],
                           preferred_element_type=jnp.float32) + b2_ref[...]


def _tc_filter(ew2b, fn1p_l, fn1_b_l, fn2_W_l, fn2_b_l):
    return pl.pallas_call(
        _filter_body,
        grid=(NEB,),
        in_specs=[
            pl.BlockSpec((8, BE), lambda e: (0, e)),
            pl.BlockSpec((NGP, H), lambda e: (0, 0)),
            pl.BlockSpec((1, H), lambda e: (0, 0)),
            pl.BlockSpec((H, H), lambda e: (0, 0)),
            pl.BlockSpec((1, H), lambda e: (0, 0)),
        ],
        out_specs=pl.BlockSpec((BE, H), lambda e: (e, 0)),
        out_shape=jax.ShapeDtypeStruct((EP, H), jnp.float32),
    )(ew2b, fn1p_l, fn1_b_l[None, :], fn2_W_l, fn2_b_l[None, :])


def _emb_body(x_ref, we_ref, be_ref, l1_ref, h_ref, hx_ref):
    h = jnp.dot(x_ref[...], we_ref[...],
                preferred_element_type=jnp.float32) + be_ref[...]
    h_ref[...] = h
    hx_ref[...] = jnp.dot(h, l1_ref[...], preferred_element_type=jnp.float32)


def _tc_emb(x, W_emb, b_emb, lin1_0):
    return pl.pallas_call(
        _emb_body,
        grid=(NNB,),
        in_specs=[
            pl.BlockSpec((BN, NODE_DIM), lambda i: (i, 0)),
            pl.BlockSpec((NODE_DIM, H), lambda i: (0, 0)),
            pl.BlockSpec((1, H), lambda i: (0, 0)),
            pl.BlockSpec((H, H), lambda i: (0, 0)),
        ],
        out_specs=[pl.BlockSpec((BN, H), lambda i: (i, 0)),
                   pl.BlockSpec((BN, H), lambda i: (i, 0))],
        out_shape=[jax.ShapeDtypeStruct((N, H), jnp.float32),
                   jax.ShapeDtypeStruct((N, H), jnp.float32)],
    )(x, W_emb, b_emb[None, :], lin1_0)


def _msg_body(hx, wf, rowp, colp, zrows, agg0, agg1,
             rows, cols, ghxs, wfvs, rss, css, gss, wss, agg_sh):
        c = lax.axis_index("c")
        s = lax.axis_index("s")
        wid = s * NC + c
        base = wid * EW

        pltpu.sync_copy(zrows, agg_sh.at[pl.ds(s * ZR, ZR)])
        plsc.subcore_barrier()

        def si(t, k):
            pltpu.async_copy(rowp.at[pl.ds(base + t * CH, CH)], rows[k],
                             rss[k])
            pltpu.async_copy(colp.at[pl.ds(base + t * CH, CH)], cols[k],
                             css[k])

        def sg(t, k, b):
            pltpu.make_async_copy(rowp.at[pl.ds(base + t * CH, CH)],
                                  rows[k], rss[k]).wait()
            pltpu.async_copy(hx.at[rows[k]], ghxs[b], gss[b])
            pltpu.async_copy(wf.at[pl.ds(base + t * CH, CH)],
                             wfvs[b], wss[b])

        def fin(t, k, b):
            ghx, wfv = ghxs[b], wfvs[b]
            pltpu.make_async_copy(hx.at[rows[k]], ghx, gss[b]).wait()
            pltpu.make_async_copy(wf.at[pl.ds(base + t * CH, CH)],
                                  wfv, wss[b]).wait()
            pltpu.make_async_copy(colp.at[pl.ds(base + t * CH, CH)],
                                  cols[k], css[k]).wait()

            @plsc.parallel_loop(0, CH, 1, unroll=4)
            def _mul(r):
                for j in range(H // 16):
                    sl = pl.ds(j * 16, 16)
                    ghx[r, sl] = ghx[r, sl] * wfv[r, sl]

            pltpu.sync_copy(ghx, agg_sh.at[cols[k]], add=True)

        si(0, 0)
        si(1, 1)
        sg(0, 0, 0)

        def step(q, _):
            t0 = q * 4
            for kk in range(4):
                t = t0 + kk

                @pl.when(t + 1 < CHUNKS)
                def _nx():
                    sg(t + 1, (kk + 1) % 4, (kk + 1) % 2)

                @pl.when(t + 2 < CHUNKS)
                def _pf():
                    si(t + 2, (kk + 2) % 4)

                fin(t, kk, kk % 2)
            return _

        lax.fori_loop(0, CHUNKS // 4, step, 0)
        plsc.subcore_barrier()

        @pl.when(c == 0)
        def _w0():
            pltpu.sync_copy(agg_sh.at[pl.ds(s * ZR, ZR)],
                            agg0.at[pl.ds(s * ZR, ZR)])

        @pl.when(c == 1)
        def _w1():
            pltpu.sync_copy(agg_sh.at[pl.ds(s * ZR, ZR)],
                            agg1.at[pl.ds(s * ZR, ZR)])


def _sc_msg(hx, wf, rowp, colp, zrows):
    return pl.kernel(
        _msg_body,
        out_type=(jax.ShapeDtypeStruct((NP, H), jnp.float32),
                  jax.ShapeDtypeStruct((NP, H), jnp.float32)),
        mesh=_MESH,
        scratch_types=[
            [pltpu.VMEM((CH,), jnp.int32) for _ in range(4)],
            [pltpu.VMEM((CH,), jnp.int32) for _ in range(4)],
            [pltpu.VMEM((CH, H), jnp.float32) for _ in range(2)],
            [pltpu.VMEM((CH, H), jnp.float32) for _ in range(2)],
            [pltpu.SemaphoreType.DMA for _ in range(4)],
            [pltpu.SemaphoreType.DMA for _ in range(4)],
            [pltpu.SemaphoreType.DMA for _ in range(2)],
            [pltpu.SemaphoreType.DMA for _ in range(2)],
            pltpu.VMEM_SHARED((NP, H), jnp.float32),
        ],
        compiler_params=pltpu.CompilerParams(needs_layout_passes=False),
        cost_estimate=pl.CostEstimate(
            flops=2 * EP * H,
            bytes_accessed=EP * (8 * H + 16) + 3 * NC * NP * H * 4,
            transcendentals=0,
        ),
    )(hx, wf, rowp, colp, zrows)


def _upd_body(a0_ref, a1_ref, h_ref, l2_ref, b2_ref, l1n_ref,
              hn_ref, hxn_ref):
    agg = a0_ref[...] + a1_ref[...]
    hn = h_ref[...] + jnp.dot(agg, l2_ref[...],
                              preferred_element_type=jnp.float32) + b2_ref[...]
    hn_ref[...] = hn
    hxn_ref[...] = jnp.dot(hn, l1n_ref[...],
                           preferred_element_type=jnp.float32)


def _tc_update(agg0, agg1, h, lin2_i, lin2_b_i, lin1_next):
    return pl.pallas_call(
        _upd_body,
        grid=(NNB,),
        in_specs=[
            pl.BlockSpec((BN, H), lambda i: (i, 0)),
            pl.BlockSpec((BN, H), lambda i: (i, 0)),
            pl.BlockSpec((BN, H), lambda i: (i, 0)),
            pl.BlockSpec((H, H), lambda i: (0, 0)),
            pl.BlockSpec((1, H), lambda i: (0, 0)),
            pl.BlockSpec((H, H), lambda i: (0, 0)),
        ],
        out_specs=[pl.BlockSpec((BN, H), lambda i: (i, 0)),
                   pl.BlockSpec((BN, H), lambda i: (i, 0))],
        out_shape=[jax.ShapeDtypeStruct((N, H), jnp.float32),
                   jax.ShapeDtypeStruct((N, H), jnp.float32)],
    )(agg0, agg1, h, lin2_i, lin2_b_i[None, :], lin1_next)


def _pool_body(batch_ref, h_ref, e3_ref, w1a_ref, w1b_ref, b1_ref,
               w2_ref, b2_ref, w3_ref, b3_ref, out_ref, acc_ref, cnt_ref):
    i = pl.program_id(0)
    nb = pl.num_programs(0)

    @pl.when(i == 0)
    def _init():
        acc_ref[...] = jnp.zeros_like(acc_ref)
        cnt_ref[...] = jnp.zeros_like(cnt_ref)

    b = batch_ref[0, 0, :]
    gids = lax.broadcasted_iota(jnp.int32, (NGRAPHS, BN), 0)
    onehot = (b[None, :] == gids).astype(jnp.float32)
    acc_ref[...] += jnp.dot(onehot, h_ref[...],
                            preferred_element_type=jnp.float32)
    cnt_ref[...] += jnp.sum(onehot, axis=1)[None, :]

    @pl.when(i == nb - 1)
    def _final():
        counts = jnp.maximum(cnt_ref[0, :], 1.0)
        scale = 1.0 / (counts * jnp.sqrt(counts))
        pooled = acc_ref[...] * scale[:, None]
        o1 = _silu(pooled @ w1a_ref[...] + e3_ref[...] @ w1b_ref[...]
                   + b1_ref[...])
        o2 = _silu(o1 @ w2_ref[...] + b2_ref[...])
        out_ref[...] = o2 @ w3_ref[...] + b3_ref[...]


def _tc_pool(h, batch, e3_row, out1_W, out1_b, out2_W, out2_b,
             out3_W, out3_b):
    batch3 = batch.reshape(NNB, 1, BN)
    return pl.pallas_call(
        _pool_body,
        grid=(NNB,),
        in_specs=[
            pl.BlockSpec((1, 1, BN), lambda i: (i, 0, 0)),
            pl.BlockSpec((BN, H), lambda i: (i, 0)),
            pl.BlockSpec((1, H), lambda i: (0, 0)),
            pl.BlockSpec((H, H), lambda i: (0, 0)),
            pl.BlockSpec((H, H), lambda i: (0, 0)),
            pl.BlockSpec((1, H), lambda i: (0, 0)),
            pl.BlockSpec((H, H // 2), lambda i: (0, 0)),
            pl.BlockSpec((1, H // 2), lambda i: (0, 0)),
            pl.BlockSpec((H // 2, 1), lambda i: (0, 0)),
            pl.BlockSpec((1, 1), lambda i: (0, 0)),
        ],
        out_specs=pl.BlockSpec((NGRAPHS, 1), lambda i: (0, 0)),
        out_shape=jax.ShapeDtypeStruct((NGRAPHS, 1), jnp.float32),
        scratch_shapes=[
            pltpu.VMEM((NGRAPHS, H), jnp.float32),
            pltpu.VMEM((1, NGRAPHS), jnp.float32),
        ],
    )(batch3, h, e3_row, out1_W[:H], out1_W[H:], out1_b[None, :],
      out2_W, out2_b[None, :], out3_W, out3_b[None, :])


def kernel(x, pos, edge_index, batch, e3_idx, W_emb, b_emb, lin1_W, lin2_W,
           lin2_b, fn1_W, fn1_b, fn2_W, fn2_b, e3_table, out1_W, out1_b,
           out2_W, out2_b, out3_W, out3_b):
    row = edge_index[0]
    col = edge_index[1]
    pad_i = jnp.arange(PAD, dtype=jnp.int32)
    rowp = jnp.concatenate([row, pad_i % N])
    colp = jnp.concatenate([col, N + (pad_i % (NP - N))])
    px = jnp.pad(pos[:, 0], (0, NPAD - N))
    py = jnp.pad(pos[:, 1], (0, NPAD - N))
    pz = jnp.pad(pos[:, 2], (0, NPAD - N))
    fn1p = jnp.pad(fn1_W, ((0, 0), (0, NGP - NG), (0, 0))).astype(jnp.bfloat16)
    fn2b16 = fn2_W.astype(jnp.bfloat16)
    zrows = jnp.zeros((ZR, H), jnp.float32)

    ew2b = _sc_geo(px, py, pz, rowp, colp)
    wfs = [_tc_filter(ew2b, fn1p[i], fn1_b[i], fn2b16[i], fn2_b[i])
           for i in range(NUM_INTER)]

    h, hx = _tc_emb(x, W_emb, b_emb, lin1_W[0])
    for i in range(NUM_INTER):
        agg0, agg1 = _sc_msg(hx, wfs[i], rowp, colp, zrows)
        lin1_next = lin1_W[(i + 1) % NUM_INTER]
        h, hx = _tc_update(agg0, agg1, h, lin2_W[i], lin2_b[i], lin1_next)

    e3_row = e3_table[e3_idx][None, :]
    return _tc_pool(h, batch, e3_row, out1_W, out1_b, out2_W, out2_b,
                    out3_W, out3_b)
